# baseline (speedup 1.0000x reference)
import functools

import jax
import jax.numpy as jnp
from jax import lax
from jax.experimental import pallas as pl
from jax.experimental.pallas import tpu as pltpu
from jax.experimental.pallas import tpu_sc as plsc

VOCAB = 50257
D = 768
B = 4
S = 2048
N = S * B
NC = 2
NS = 16
NW = NC * NS
RPW = N // NW
SPW = S // NW
NCH = 16
C = RPW // NCH
CS = C // B
LANES = 16
NDB = D // LANES
NGB = 4
NPB = 4
NSB = 2
LOOK = 2


def _sc_embed(idx3, wte, wpe):
    mesh = plsc.VectorSubcoreMesh(core_axis_name="c", subcore_axis_name="s")

    @functools.partial(
        pl.kernel,
        mesh=mesh,
        out_type=jax.ShapeDtypeStruct((S, B, D), jnp.float32),
        scratch_types=(
            [pltpu.VMEM((NCH, C), jnp.int32)]
            + [pltpu.VMEM((C, D), jnp.float32)] * NGB
            + [pltpu.VMEM((CS, B, D), jnp.float32)] * NSB
            + [pltpu.VMEM((CS, D), jnp.float32)] * NPB
            + [pltpu.SemaphoreType.DMA] * (NGB + NSB + NPB)
        ),
    )
    def k(idx_hbm, wte_hbm, wpe_hbm, out_hbm,
          idx_v, gb0, gb1, gb2, gb3, sb0, sb1, pe0, pe1, pe2, pe3,
          g0, g1, g2, g3, w0, w1, q0, q1, q2, q3):
        gbufs = (gb0, gb1, gb2, gb3)
        sbufs = (sb0, sb1)
        pbufs = (pe0, pe1, pe2, pe3)
        gsems = (g0, g1, g2, g3)
        wsems = (w0, w1)
        qsems = (q0, q1, q2, q3)
        wid = lax.axis_index("s") * NC + lax.axis_index("c")
        s0 = wid * SPW
        pltpu.sync_copy(idx_hbm.at[wid], idx_v)
        gdescs = [None] * NGB
        pdescs = [None] * NPB
        wdescs = [None] * NSB
        for j in range(LOOK):
            pdescs[j] = pltpu.async_copy(
                wpe_hbm.at[pl.ds(s0 + j * CS, CS)], pbufs[j], qsems[j])
            gdescs[j] = pltpu.async_copy(
                wte_hbm.at[idx_v.at[j]], gbufs[j], gsems[j])
        for j in range(NCH):
            gi = j % NGB
            si = j % NSB
            pi = j % NPB
            jl = j + LOOK
            if jl < NCH:
                gdescs[jl % NGB] = pltpu.async_copy(
                    wte_hbm.at[idx_v.at[jl]], gbufs[jl % NGB],
                    gsems[jl % NGB])
                pdescs[jl % NPB] = pltpu.async_copy(
                    wpe_hbm.at[pl.ds(s0 + jl * CS, CS)], pbufs[jl % NPB],
                    qsems[jl % NPB])
            gdescs[gi].wait()
            pdescs[pi].wait()
            if j >= NSB:
                wdescs[si].wait()
            gbuf = gbufs[gi]
            sbuf = sbufs[si]
            pbuf = pbufs[pi]

            def add_body(i, carry, gbuf=gbuf, sbuf=sbuf, pbuf=pbuf):
                off = i * LANES
                for sl in range(CS):
                    w = pbuf[sl, pl.ds(off, LANES)]
                    for bb in range(B):
                        r = sl * B + bb
                        sbuf[sl, bb, pl.ds(off, LANES)] = (
                            gbuf[r, pl.ds(off, LANES)] + w
                        )
                return carry

            lax.fori_loop(0, NDB, add_body, 0)
            wdescs[si] = pltpu.async_copy(
                sbuf, out_hbm.at[pl.ds(s0 + j * CS, CS)], wsems[si])
        for j in range(NCH - NSB, NCH):
            wdescs[j % NSB].wait()

    return k(idx3, wte, wpe)


def kernel(input_ids, wte, wpe):
    idx3 = jnp.transpose(input_ids).reshape(NW, NCH, C)
    hidden = _sc_embed(idx3, wte, wpe)
    return (hidden, input_ids)
